# baseline (speedup 1.0000x reference)
import functools

import jax
import jax.numpy as jnp
from jax import lax
from jax.experimental import pallas as pl
from jax.experimental.pallas import tpu as pltpu
from jax.experimental.pallas import tpu_sc as plsc

B = 16384
D = 51
NC = 2
NS = 16
L = 16
NW = NC * NS
BPW = B // NW
GROUPS = BPW // L

_mesh = plsc.VectorSubcoreMesh(
    core_axis_name="c", subcore_axis_name="s", num_cores=NC, num_subcores=NS
)


@functools.partial(
    pl.kernel,
    out_type=jax.ShapeDtypeStruct((B, D), jnp.float32),
    mesh=_mesh,
    scratch_types=[
        pltpu.VMEM((BPW,), jnp.float32),
        pltpu.VMEM((BPW,), jnp.int32),
        pltpu.VMEM((BPW,), jnp.float32),
        pltpu.VMEM((2 * BPW, D), jnp.float32),
        pltpu.VMEM((BPW, D), jnp.float32),
        pltpu.SemaphoreType.DMA,
    ],
    compiler_params=pltpu.CompilerParams(
        needs_layout_passes=False, use_tc_tiling_on_sc=False
    ),
)
def _sc_interp(x_hbm, grid_hbm, out_hbm, x_v, idx_v, w_v, r_v, o_v, sem):
    wid = lax.axis_index("s") * NC + lax.axis_index("c")
    base = wid * BPW

    pltpu.sync_copy(x_hbm.at[pl.ds(base, BPW)], x_v)

    for i in range(GROUPS):
        sl = pl.ds(i * L, L)
        xv = x_v[sl]
        x0 = xv.astype(jnp.int32)
        idx_v[sl] = x0
        w_v[sl] = xv - x0.astype(jnp.float32)

    def enqueue(g, carry):
        iv = idx_v[pl.ds(g * L, L)]
        for k in range(L):
            s = g * L + k
            pltpu.async_copy(
                grid_hbm.at[pl.ds(iv[k], 2)], r_v.at[pl.ds(2 * s, 2)], sem
            )
        return carry

    lax.fori_loop(0, GROUPS, enqueue, 0)
    pltpu.make_async_copy(out_hbm.at[pl.ds(0, 2 * BPW)], r_v, sem).wait()

    def body(g, carry):
        row0 = (g * L + lax.iota(jnp.int32, L)) * 2
        row1 = row0 + 1
        wv = w_v[pl.ds(g * L, L)]
        for j in range(D):
            col = jnp.full((L,), j, jnp.int32)
            a = plsc.load_gather(r_v, [row0, col])
            b = plsc.load_gather(r_v, [row1, col])
            o = (a + wv * (b - a)) * 1000.0
            plsc.store_scatter(o_v, [g * L + lax.iota(jnp.int32, L), col], o)
        return carry

    lax.fori_loop(0, GROUPS, body, 0)

    pltpu.sync_copy(o_v, out_hbm.at[pl.ds(base, BPW)])


def kernel(x, grid):
    return _sc_interp(x, grid).reshape(B, 17, 3)


# speedup vs baseline: 2.4224x; 2.4224x over previous
import functools

import jax
import jax.numpy as jnp
from jax import lax
from jax.experimental import pallas as pl
from jax.experimental.pallas import tpu as pltpu
from jax.experimental.pallas import tpu_sc as plsc

B = 16384
D = 51
NROWS = 1000001
NC = 2
NS = 16
L = 16
NW = NC * NS
BPW = B // NW
GROUPS = BPW // L
CH = 32
CG = CH // L
NCHUNK = BPW // CH
W = 16
TMAX = ((NROWS - W) // 8) * 8

_mesh = plsc.VectorSubcoreMesh(
    core_axis_name="c", subcore_axis_name="s", num_cores=NC, num_subcores=NS
)


@functools.partial(
    pl.kernel,
    out_type=jax.ShapeDtypeStruct((B * D,), jnp.float32),
    mesh=_mesh,
    scratch_types=[
        pltpu.VMEM((BPW,), jnp.float32),
        pltpu.VMEM((BPW,), jnp.int32),
        pltpu.VMEM((BPW,), jnp.float32),
        pltpu.VMEM((CH * W, D), jnp.float32),
        pltpu.VMEM((BPW * D,), jnp.float32),
        pltpu.SemaphoreType.DMA,
    ],
    compiler_params=pltpu.CompilerParams(
        needs_layout_passes=False, use_tc_tiling_on_sc=True
    ),
)
def _sc_interp(x_hbm, grid_hbm, out_hbm, x_v, t8_v, w_v, t_v, o_v, sem):
    wid = lax.axis_index("s") * NC + lax.axis_index("c")
    base = wid * BPW

    pltpu.sync_copy(x_hbm.at[pl.ds(base, BPW)], x_v)

    for i in range(GROUPS):
        sl = pl.ds(i * L, L)
        xv = x_v[sl]
        x0 = xv.astype(jnp.int32)
        t8_v[sl] = jnp.minimum((x0 >> 3) << 3, TMAX)
        w_v[sl] = xv - x0.astype(jnp.float32)

    def chunk_body(c, carry):
        cbase = c * CH
        for g in range(CG):
            tv = t8_v[pl.ds(cbase + g * L, L)]
            for k in range(L):
                sl_local = g * L + k
                pltpu.async_copy(
                    grid_hbm.at[pl.ds(pl.multiple_of(tv[k], 8), W)],
                    t_v.at[pl.ds(sl_local * W, W)],
                    sem,
                )
        pltpu.make_async_copy(grid_hbm.at[pl.ds(0, CH * W)], t_v, sem).wait()

        for g in range(CG):
            sl = pl.ds(cbase + g * L, L)
            x0 = x_v[sl].astype(jnp.int32)
            krow = x0 - t8_v[sl]
            rowa = (g * L + lax.iota(jnp.int32, L)) * W + krow
            wv = w_v[sl]
            oflat = (cbase + g * L + lax.iota(jnp.int32, L)) * D
            for j in range(D):
                col = jnp.full((L,), j, jnp.int32)
                a = plsc.load_gather(t_v, [rowa, col])
                b = plsc.load_gather(t_v, [rowa + 1, col])
                o = (a + wv * (b - a)) * 1000.0
                plsc.store_scatter(o_v, [oflat + j], o)
        return carry

    lax.fori_loop(0, NCHUNK, chunk_body, 0)

    pltpu.sync_copy(o_v, out_hbm.at[pl.ds(base * D, BPW * D)])


def kernel(x, grid):
    return _sc_interp(x, grid).reshape(B, 17, 3)


# speedup vs baseline: 3.7231x; 1.5370x over previous
import functools

import jax
import jax.numpy as jnp
from jax import lax
from jax.experimental import pallas as pl
from jax.experimental.pallas import tpu as pltpu
from jax.experimental.pallas import tpu_sc as plsc

B = 16384
D = 51
NROWS = 1000001
NC = 2
NS = 16
L = 16
NW = NC * NS
BPW = B // NW
GROUPS = BPW // L
CH = 32
CG = CH // L
NCHUNK = BPW // CH
W = 16
TMAX = ((NROWS - W) // 8) * 8

_mesh = plsc.VectorSubcoreMesh(
    core_axis_name="c", subcore_axis_name="s", num_cores=NC, num_subcores=NS
)


@functools.partial(
    pl.kernel,
    out_type=jax.ShapeDtypeStruct((D * B,), jnp.float32),
    mesh=_mesh,
    scratch_types=[
        pltpu.VMEM((BPW,), jnp.float32),
        pltpu.VMEM((BPW,), jnp.int32),
        pltpu.VMEM((BPW,), jnp.float32),
        pltpu.VMEM((CH * W, D), jnp.float32),
        pltpu.VMEM((CH * W, D), jnp.float32),
        pltpu.VMEM((D * BPW,), jnp.float32),
        pltpu.SemaphoreType.DMA,
        pltpu.SemaphoreType.DMA,
    ],
    compiler_params=pltpu.CompilerParams(
        needs_layout_passes=False, use_tc_tiling_on_sc=True
    ),
)
def _sc_interp(x_hbm, grid_hbm, out_hbm, x_v, t8_v, w_v, ta_v, tb_v, o_v, sema, semb):
    wid = lax.axis_index("s") * NC + lax.axis_index("c")
    base = wid * BPW

    pltpu.sync_copy(x_hbm.at[pl.ds(base, BPW)], x_v)

    for i in range(GROUPS):
        sl = pl.ds(i * L, L)
        xv = x_v[sl]
        x0 = xv.astype(jnp.int32)
        t8_v[sl] = jnp.minimum((x0 >> 3) << 3, TMAX)
        w_v[sl] = xv - x0.astype(jnp.float32)

    def fetch(c, t_v, sem):
        cbase = c * CH
        for g in range(CG):
            tv = t8_v[pl.ds(cbase + g * L, L)]
            for k in range(L):
                sl_local = g * L + k
                pltpu.async_copy(
                    grid_hbm.at[pl.ds(pl.multiple_of(tv[k], 8), W)],
                    t_v.at[pl.ds(sl_local * W, W)],
                    sem,
                )

    def drain(t_v, sem):
        pltpu.make_async_copy(grid_hbm.at[pl.ds(0, CH * W)], t_v, sem).wait()

    def lerp(c, t_v):
        cbase = c * CH
        for g in range(CG):
            sl = pl.ds(cbase + g * L, L)
            x0 = x_v[sl].astype(jnp.int32)
            krow = x0 - t8_v[sl]
            rowa = (g * L + lax.iota(jnp.int32, L)) * W + krow
            wv = w_v[sl]
            oflat = cbase + g * L + lax.iota(jnp.int32, L)
            for j in range(D):
                col = jnp.full((L,), j, jnp.int32)
                a = plsc.load_gather(t_v, [rowa, col])
                b = plsc.load_gather(t_v, [rowa + 1, col])
                o = (a + wv * (b - a)) * 1000.0
                plsc.store_scatter(o_v, [oflat + j * BPW], o)

    def pipe(c, carry):
        fetch(c, ta_v, sema)
        drain(ta_v, sema)
        lerp(c, ta_v)
        return carry

    lax.fori_loop(0, NCHUNK, pipe, 0)

    pltpu.sync_copy(o_v, out_hbm.at[pl.ds(wid * (D * BPW), D * BPW)])


def kernel(x, grid):
    flat = _sc_interp(x, grid)
    return flat.reshape(NW, 17, 3, BPW).transpose(0, 3, 1, 2).reshape(B, 17, 3)


# speedup vs baseline: 3.7412x; 1.0049x over previous
import functools

import jax
import jax.numpy as jnp
from jax import lax
from jax.experimental import pallas as pl
from jax.experimental.pallas import tpu as pltpu
from jax.experimental.pallas import tpu_sc as plsc

B = 16384
D = 51
NROWS = 1000001
NC = 2
NS = 16
L = 16
NW = NC * NS
BPW = B // NW
GROUPS = BPW // L
CH = 32
CG = CH // L
NCHUNK = BPW // CH
W = 16
TMAX = ((NROWS - W) // 8) * 8

_mesh = plsc.VectorSubcoreMesh(
    core_axis_name="c", subcore_axis_name="s", num_cores=NC, num_subcores=NS
)


@functools.partial(
    pl.kernel,
    out_type=jax.ShapeDtypeStruct((D * B,), jnp.float32),
    mesh=_mesh,
    scratch_types=[
        pltpu.VMEM((BPW,), jnp.float32),
        pltpu.VMEM((BPW,), jnp.int32),
        pltpu.VMEM((BPW,), jnp.float32),
        pltpu.VMEM((CH * W, D), jnp.float32),
        pltpu.VMEM((D * BPW,), jnp.float32),
        pltpu.SemaphoreType.DMA,
    ],
    compiler_params=pltpu.CompilerParams(
        needs_layout_passes=False, use_tc_tiling_on_sc=True
    ),
)
def _sc_interp(x_hbm, grid_hbm, out_hbm, x_v, t8_v, w_v, ta_v, o_v, sema):
    wid = lax.axis_index("s") * NC + lax.axis_index("c")
    base = wid * BPW

    pltpu.sync_copy(x_hbm.at[pl.ds(base, BPW)], x_v)

    for i in range(GROUPS):
        sl = pl.ds(i * L, L)
        xv = x_v[sl]
        x0 = xv.astype(jnp.int32)
        t8_v[sl] = jnp.minimum((x0 >> 3) << 3, TMAX)
        w_v[sl] = xv - x0.astype(jnp.float32)

    def fetch(c, t_v, sem):
        cbase = c * CH
        for g in range(CG):
            tv = t8_v[pl.ds(cbase + g * L, L)]
            for k in range(L):
                sl_local = g * L + k
                pltpu.async_copy(
                    grid_hbm.at[pl.ds(pl.multiple_of(tv[k], 8), W)],
                    t_v.at[pl.ds(sl_local * W, W)],
                    sem,
                )

    def drain(t_v, sem):
        pltpu.make_async_copy(grid_hbm.at[pl.ds(0, CH * W)], t_v, sem).wait()

    def lerp(c, t_v):
        cbase = c * CH
        for g in range(CG):
            sl = pl.ds(cbase + g * L, L)
            x0 = x_v[sl].astype(jnp.int32)
            krow = x0 - t8_v[sl]
            rowa = (g * L + lax.iota(jnp.int32, L)) * W + krow
            wv = w_v[sl]
            oflat = cbase + g * L + lax.iota(jnp.int32, L)

            def jbody(j, carry):
                col = jnp.full((L,), j, jnp.int32)
                a = plsc.load_gather(t_v, [rowa, col])
                b = plsc.load_gather(t_v, [rowa + 1, col])
                o = (a + wv * (b - a)) * 1000.0
                plsc.store_scatter(o_v, [oflat + j * BPW], o)
                return carry

            lax.fori_loop(0, D, jbody, 0)

    def pipe(c, carry):
        fetch(c, ta_v, sema)
        drain(ta_v, sema)
        lerp(c, ta_v)
        return carry

    lax.fori_loop(0, NCHUNK, pipe, 0)

    pltpu.sync_copy(o_v, out_hbm.at[pl.ds(wid * (D * BPW), D * BPW)])


def kernel(x, grid):
    flat = _sc_interp(x, grid)
    return flat.reshape(NW, 17, 3, BPW).transpose(0, 3, 1, 2).reshape(B, 17, 3)


# speedup vs baseline: 3.9095x; 1.0450x over previous
import functools

import jax
import jax.numpy as jnp
from jax import lax
from jax.experimental import pallas as pl
from jax.experimental.pallas import tpu as pltpu
from jax.experimental.pallas import tpu_sc as plsc

B = 16384
D = 51
NROWS = 1000001
NC = 2
NS = 16
L = 16
NW = NC * NS
BPW = B // NW
GROUPS = BPW // L
CH = 32
CG = CH // L
NCHUNK = BPW // CH
W = 16
TMAX = ((NROWS - W) // 8) * 8

_mesh = plsc.VectorSubcoreMesh(
    core_axis_name="c", subcore_axis_name="s", num_cores=NC, num_subcores=NS
)


@functools.partial(
    pl.kernel,
    out_type=jax.ShapeDtypeStruct((D * B,), jnp.float32),
    mesh=_mesh,
    scratch_types=[
        pltpu.VMEM((BPW,), jnp.float32),
        pltpu.VMEM((BPW,), jnp.int32),
        pltpu.VMEM((BPW,), jnp.float32),
        pltpu.VMEM((CH * W, D), jnp.float32),
        pltpu.VMEM((D * BPW,), jnp.float32),
        pltpu.SemaphoreType.DMA,
    ],
    compiler_params=pltpu.CompilerParams(
        needs_layout_passes=False, use_tc_tiling_on_sc=True
    ),
)
def _sc_interp(x_hbm, grid_hbm, out_hbm, x_v, t8_v, w_v, ta_v, o_v, sema):
    wid = lax.axis_index("s") * NC + lax.axis_index("c")
    base = wid * BPW

    pltpu.sync_copy(x_hbm.at[pl.ds(base, BPW)], x_v)

    for i in range(GROUPS):
        sl = pl.ds(i * L, L)
        xv = x_v[sl]
        x0 = xv.astype(jnp.int32)
        t8_v[sl] = jnp.minimum((x0 >> 3) << 3, TMAX)
        w_v[sl] = xv - x0.astype(jnp.float32)

    def fetch(c, t_v, sem):
        cbase = c * CH
        n_extra = jnp.int32(0)
        for g in range(CG):
            sl = pl.ds(cbase + g * L, L)
            tv = t8_v[sl]
            xv0 = x_v[sl].astype(jnp.int32)
            spill = xv0 + 1 - tv >= 8
            spilli = spill.astype(jnp.int32)
            n_extra = n_extra + plsc.all_reduce_population_count(spill)[0]
            for k in range(L):
                sl_local = g * L + k
                base8 = pl.multiple_of(tv[k], 8)
                pltpu.async_copy(
                    grid_hbm.at[pl.ds(base8, 8)],
                    t_v.at[pl.ds(sl_local * W, 8)],
                    sem,
                )

                @pl.when(spilli[k] == 1)
                def _():
                    pltpu.async_copy(
                        grid_hbm.at[pl.ds(pl.multiple_of(tv[k] + 8, 8), 8)],
                        t_v.at[pl.ds(sl_local * W + 8, 8)],
                        sem,
                    )

        return n_extra

    def drain(t_v, sem, n_extra):
        pltpu.make_async_copy(
            grid_hbm.at[pl.ds(0, CH * 8)], t_v.at[pl.ds(0, CH * 8)], sem
        ).wait()

        def dbody(i, carry):
            pltpu.make_async_copy(
                grid_hbm.at[pl.ds(0, 8)], t_v.at[pl.ds(0, 8)], sem
            ).wait()
            return carry

        lax.fori_loop(0, n_extra, dbody, 0)

    def lerp(c, t_v):
        cbase = c * CH
        for g in range(CG):
            sl = pl.ds(cbase + g * L, L)
            x0 = x_v[sl].astype(jnp.int32)
            krow = x0 - t8_v[sl]
            rowa = (g * L + lax.iota(jnp.int32, L)) * W + krow
            wv = w_v[sl]
            oflat = cbase + g * L + lax.iota(jnp.int32, L)

            def jbody(j, carry):
                col = jnp.full((L,), j, jnp.int32)
                a = plsc.load_gather(t_v, [rowa, col])
                b = plsc.load_gather(t_v, [rowa + 1, col])
                o = (a + wv * (b - a)) * 1000.0
                plsc.store_scatter(o_v, [oflat + j * BPW], o)
                return carry

            lax.fori_loop(0, D, jbody, 0)

    def pipe(c, carry):
        n_extra = fetch(c, ta_v, sema)
        drain(ta_v, sema, n_extra)
        lerp(c, ta_v)
        return carry

    lax.fori_loop(0, NCHUNK, pipe, 0)

    pltpu.sync_copy(o_v, out_hbm.at[pl.ds(wid * (D * BPW), D * BPW)])


def kernel(x, grid):
    flat = _sc_interp(x, grid)
    return flat.reshape(NW, 17, 3, BPW).transpose(0, 3, 1, 2).reshape(B, 17, 3)
